# Initial kernel scaffold; baseline (speedup 1.0000x reference)
import jax
import jax.numpy as jnp
from jax.experimental import pallas as pl


def kernel(static, dynamic, station_num_lim, W, v):
    raise NotImplementedError("write your pallas kernel here")


# speedup vs baseline: 2.1317x; 2.1317x over previous
import functools

import jax
import jax.numpy as jnp
from jax.experimental import pallas as pl
from jax.experimental.pallas import tpu as pltpu

N = 1_000_000
NP = 1 << 20
ROWS, COLS = 1024, 1024
BLK_ROWS = 128
NBLK = ROWS // BLK_ROWS
H = 16
NEG = -1e30
STEPS = 16


def _bf(x):
    return x.astype(jnp.bfloat16).astype(jnp.float32)


def _score_body(a_ref, b_ref, v_ref, x_ref, y_ref, s_ref, stat_ref):
    bi = pl.program_id(0)
    x = _bf(x_ref[...])
    y = _bf(y_ref[...])
    s = jnp.zeros_like(x)
    for j in range(H):
        z = a_ref[j] * x + b_ref[j] * y
        s = s + v_ref[j] * _bf(jnp.tanh(z))
    r = jax.lax.broadcasted_iota(jnp.int32, (BLK_ROWS, COLS), 0)
    c = jax.lax.broadcasted_iota(jnp.int32, (BLK_ROWS, COLS), 1)
    gidx = (bi * BLK_ROWS + r) * COLS + c
    s = jnp.where(gidx < N, s, NEG)
    s_ref[...] = s
    m = jnp.max(s)
    se = jnp.sum(jnp.exp(s - m))
    lane = jax.lax.broadcasted_iota(jnp.int32, (1, 1, 128), 2)
    stat_ref[...] = jnp.where(lane == 0, m, jnp.where(lane == 1, se, 0.0))


def _scores(static, W, v):
    x = jnp.pad(static[0, 0, :], (0, NP - N)).reshape(ROWS, COLS)
    y = jnp.pad(static[0, 1, :], (0, NP - N)).reshape(ROWS, COLS)
    a = jax.lax.reduce_precision(W[:, 0], exponent_bits=8, mantissa_bits=7)
    b = jax.lax.reduce_precision(W[:, 1], exponent_bits=8, mantissa_bits=7)
    v = jax.lax.reduce_precision(v, exponent_bits=8, mantissa_bits=7)
    smem_spec = pl.BlockSpec(memory_space=pltpu.SMEM)
    s0, stats = pl.pallas_call(
        _score_body,
        grid=(NBLK,),
        in_specs=[
            smem_spec,
            smem_spec,
            smem_spec,
            pl.BlockSpec((BLK_ROWS, COLS), lambda i: (i, 0)),
            pl.BlockSpec((BLK_ROWS, COLS), lambda i: (i, 0)),
        ],
        out_specs=[
            pl.BlockSpec((BLK_ROWS, COLS), lambda i: (i, 0)),
            pl.BlockSpec((1, 1, 128), lambda i: (i, 0, 0)),
        ],
        out_shape=[
            jax.ShapeDtypeStruct((ROWS, COLS), jnp.float32),
            jax.ShapeDtypeStruct((NBLK, 1, 128), jnp.float32),
        ],
    )(a, b, v, x, y)
    return s0, stats


def kernel(static, dynamic, station_num_lim, W, v):
    s0, stats = _scores(static, W, v)
    mb = stats[:, 0, 0]
    sb = stats[:, 0, 1]
    M = jnp.max(mb)
    S0 = jnp.sum(jnp.exp(mb - M) * sb)
    flat = s0.reshape(-1)
    vals, idx = jax.lax.top_k(flat, STEPS)
    ex = jnp.exp(vals - M)
    removed = jnp.concatenate([jnp.zeros((1,), jnp.float32), jnp.cumsum(ex)[:-1]])
    logp = vals - M - jnp.log(S0 - removed)
    return idx[None, :].astype(jnp.int32), logp[None, :]


# speedup vs baseline: 17.3604x; 8.1439x over previous
import functools

import jax
import jax.numpy as jnp
from jax import lax
from jax.experimental import pallas as pl
from jax.experimental.pallas import tpu as pltpu
from jax.experimental.pallas import tpu_sc as plsc

N = 1_000_000
NP = 1 << 20
ROWS, COLS = 1024, 1024
BLK_ROWS = 128
NBLK = ROWS // BLK_ROWS
H = 16
NEG = -1e30
STEPS = 16

NWORK = 32
SHARD = NP // NWORK
NGRP = 16
GRP = SHARD // NGRP
GVEC = GRP // 16
BIG = 0x7FFFFFFF


def _bf(x):
    return x.astype(jnp.bfloat16).astype(jnp.float32)


def _score_body(a_ref, b_ref, v_ref, x_ref, y_ref, s_ref, stat_ref):
    bi = pl.program_id(0)
    x = _bf(x_ref[...])
    y = _bf(y_ref[...])
    s = jnp.zeros_like(x)
    for j in range(H):
        z = a_ref[j] * x + b_ref[j] * y
        s = s + v_ref[j] * _bf(jnp.tanh(z))
    r = jax.lax.broadcasted_iota(jnp.int32, (BLK_ROWS, COLS), 0)
    c = jax.lax.broadcasted_iota(jnp.int32, (BLK_ROWS, COLS), 1)
    gidx = (bi * BLK_ROWS + r) * COLS + c
    s = jnp.where(gidx < N, s, NEG)
    s_ref[...] = s
    m = jnp.max(s)
    se = jnp.sum(jnp.exp(s - m))
    lane = jax.lax.broadcasted_iota(jnp.int32, (1, 1, 128), 2)
    stat_ref[...] = jnp.where(lane == 0, m, jnp.where(lane == 1, se, 0.0))


def _scores(static, W, v):
    x = jnp.pad(static[0, 0, :], (0, NP - N)).reshape(ROWS, COLS)
    y = jnp.pad(static[0, 1, :], (0, NP - N)).reshape(ROWS, COLS)
    a = jax.lax.reduce_precision(W[:, 0], exponent_bits=8, mantissa_bits=7)
    b = jax.lax.reduce_precision(W[:, 1], exponent_bits=8, mantissa_bits=7)
    v = jax.lax.reduce_precision(v, exponent_bits=8, mantissa_bits=7)
    smem_spec = pl.BlockSpec(memory_space=pltpu.SMEM)
    s0, stats = pl.pallas_call(
        _score_body,
        grid=(NBLK,),
        in_specs=[
            smem_spec,
            smem_spec,
            smem_spec,
            pl.BlockSpec((BLK_ROWS, COLS), lambda i: (i, 0)),
            pl.BlockSpec((BLK_ROWS, COLS), lambda i: (i, 0)),
        ],
        out_specs=[
            pl.BlockSpec((BLK_ROWS, COLS), lambda i: (i, 0)),
            pl.BlockSpec((1, 1, 128), lambda i: (i, 0, 0)),
        ],
        out_shape=[
            jax.ShapeDtypeStruct((ROWS, COLS), jnp.float32),
            jax.ShapeDtypeStruct((NBLK, 1, 128), jnp.float32),
        ],
    )(a, b, v, x, y)
    return s0, stats


def _lperm(x, idx):
    dn = lax.GatherDimensionNumbers(
        offset_dims=(), collapsed_slice_dims=(0,), start_index_map=(0,))
    return lax.gather(x, idx.reshape(16, 1), dn, slice_sizes=(1,),
                      mode=lax.GatherScatterMode.PROMISE_IN_BOUNDS)


def _allmax(x, lane):
    for s in (1, 2, 4, 8):
        x = jnp.maximum(x, _lperm(x, lane ^ s))
    return x


def _allmin(x, lane):
    for s in (1, 2, 4, 8):
        x = jnp.minimum(x, _lperm(x, lane ^ s))
    return x


def _sc_topk_body(s0_hbm, vals_hbm, idx_hbm, data_v, vrow_v, irow_v):
    wid = lax.axis_index("s") * 2 + lax.axis_index("c")
    pltpu.sync_copy(s0_hbm.at[pl.ds(wid * SHARD, SHARD)], data_v)

    lane = lax.iota(jnp.int32, 16)
    fneg = jnp.full((16,), -3.0e38, jnp.float32)

    def group_scan(g_base):
        def body(i, carry):
            v1, v2, i1 = carry
            base = g_base + i * 16
            x = data_v[pl.ds(base, 16)]
            idxv = base + lane
            upd = x > v1
            v2 = jnp.where(upd, v1, jnp.where(x > v2, x, v2))
            i1 = jnp.where(upd, idxv, i1)
            v1 = jnp.where(upd, x, v1)
            return v1, v2, i1

        return lax.fori_loop(
            0, GVEC, body, (fneg, fneg, jnp.full((16,), BIG, jnp.int32)))

    sg = fneg
    for g in range(NGRP):
        v1, _, _ = group_scan(jnp.int32(g * GRP))
        sg = jnp.where(lane == g, _allmax(v1, lane), sg)

    vrow = fneg
    irow = jnp.zeros((16,), jnp.int32)
    for t in range(STEPS):
        mtv = _allmax(sg, lane)
        gsv = _allmin(jnp.where(sg == mtv, lane, BIG), lane)
        v1, v2, i1 = group_scan(gsv[0] * GRP)
        mgv = _allmax(v1, lane)
        posv = _allmin(jnp.where(v1 == mgv, i1, BIG), lane)
        pos = posv[0]
        wl = pos & 15
        wbase = pos - wl
        w = data_v[pl.ds(wbase, 16)]
        data_v[pl.ds(wbase, 16)] = jnp.where(lane == wl, NEG, w)
        sg_new = _allmax(jnp.where(lane == wl, v2, v1), lane)
        sg = jnp.where(lane == gsv, sg_new, sg)
        vrow = jnp.where(lane == t, mgv, vrow)
        irow = jnp.where(lane == t, posv + wid * SHARD, irow)

    vrow_v[...] = vrow
    irow_v[...] = irow
    pltpu.sync_copy(vrow_v, vals_hbm.at[wid])
    pltpu.sync_copy(irow_v, idx_hbm.at[wid])


def _sc_topk(s0_flat):
    mesh = plsc.VectorSubcoreMesh(core_axis_name="c", subcore_axis_name="s")
    kfn = functools.partial(
        pl.kernel,
        mesh=mesh,
        out_type=[
            jax.ShapeDtypeStruct((NWORK, 16), jnp.float32),
            jax.ShapeDtypeStruct((NWORK, 16), jnp.int32),
        ],
        scratch_types=[
            pltpu.VMEM((SHARD,), jnp.float32),
            pltpu.VMEM((16,), jnp.float32),
            pltpu.VMEM((16,), jnp.int32),
        ],
    )(_sc_topk_body)
    return kfn(s0_flat)


def _merge_body(v_ref, i_ref, st_ref, ti_ref, lp_ref):
    V = v_ref[...]
    I = i_ref[...]
    st = st_ref[...]
    i2 = jax.lax.broadcasted_iota(jnp.int32, (NBLK, 1, 128), 2)
    mcol = jnp.max(jnp.where(i2 == 0, st, NEG), axis=2, keepdims=True)
    scol = jnp.sum(jnp.where(i2 == 1, st, 0.0), axis=2, keepdims=True)
    M = jnp.max(mcol)
    S = jnp.sum(jnp.exp(mcol - M) * scol)
    lane16 = jax.lax.broadcasted_iota(jnp.int32, (1, 16), 1)
    ti = jnp.zeros((1, 16), jnp.int32)
    lp = jnp.zeros((1, 16), jnp.float32)
    for t in range(STEPS):
        cur = jnp.max(V)
        pick = jnp.min(jnp.where(V == cur, I, BIG))
        V = jnp.where((V == cur) & (I == pick), NEG, V)
        lp = jnp.where(lane16 == t, cur - M - jnp.log(S), lp)
        ti = jnp.where(lane16 == t, pick, ti)
        S = S - jnp.exp(cur - M)
    ti_ref[...] = ti
    lp_ref[...] = lp


def _merge(cand_v, cand_i, stats):
    return pl.pallas_call(
        _merge_body,
        out_shape=[
            jax.ShapeDtypeStruct((1, STEPS), jnp.int32),
            jax.ShapeDtypeStruct((1, STEPS), jnp.float32),
        ],
    )(cand_v, cand_i, stats)


def kernel(static, dynamic, station_num_lim, W, v):
    s0, stats = _scores(static, W, v)
    cand_v, cand_i = _sc_topk(s0.reshape(-1))
    tour_idx, tour_logp = _merge(cand_v, cand_i, stats)
    return tour_idx, tour_logp


# speedup vs baseline: 24.7457x; 1.4254x over previous
import functools

import jax
import jax.numpy as jnp
from jax import lax
from jax.experimental import pallas as pl
from jax.experimental.pallas import tpu as pltpu
from jax.experimental.pallas import tpu_sc as plsc

N = 1_000_000
NP = 1 << 20
ROWS, COLS = 1024, 1024
BLK_ROWS = 128
NBLK = ROWS // BLK_ROWS
H = 16
NEG = -1e30
STEPS = 16

NWORK = 32
SHARD = NP // NWORK
NGRP = 16
GRP = SHARD // NGRP
GVEC = GRP // 16
BIG = 0x7FFFFFFF


def _bf(x):
    return x.astype(jnp.bfloat16).astype(jnp.float32)


def _score_body(a_ref, b_ref, v_ref, x_ref, y_ref, s_ref, stat_ref):
    bi = pl.program_id(0)
    x = _bf(x_ref[0])
    y = _bf(y_ref[0])
    s = jnp.zeros_like(x)
    for j in range(H):
        z = a_ref[j] * x + b_ref[j] * y
        s = s + v_ref[j] * _bf(jnp.tanh(z))
    r = jax.lax.broadcasted_iota(jnp.int32, (BLK_ROWS, COLS), 0)
    c = jax.lax.broadcasted_iota(jnp.int32, (BLK_ROWS, COLS), 1)
    gidx = (bi * BLK_ROWS + r) * COLS + c
    s = jnp.where(gidx < N, s, NEG)
    s_ref[...] = s
    m = jnp.max(s)
    se = jnp.sum(jnp.exp(s - m))
    lane = jax.lax.broadcasted_iota(jnp.int32, (1, 1, 128), 2)
    stat_ref[...] = jnp.where(lane == 0, m, jnp.where(lane == 1, se, 0.0))


def _scores(static, W, v):
    xy = jnp.concatenate(
        [static.reshape(2, N), jnp.zeros((2, NP - N), jnp.float32)], axis=1
    ).reshape(2, ROWS, COLS)
    a = jax.lax.reduce_precision(W[:, 0], exponent_bits=8, mantissa_bits=7)
    b = jax.lax.reduce_precision(W[:, 1], exponent_bits=8, mantissa_bits=7)
    v = jax.lax.reduce_precision(v, exponent_bits=8, mantissa_bits=7)
    smem_spec = pl.BlockSpec(memory_space=pltpu.SMEM)
    s0, stats = pl.pallas_call(
        _score_body,
        grid=(NBLK,),
        in_specs=[
            smem_spec,
            smem_spec,
            smem_spec,
            pl.BlockSpec((1, BLK_ROWS, COLS), lambda i: (0, i, 0)),
            pl.BlockSpec((1, BLK_ROWS, COLS), lambda i: (1, i, 0)),
        ],
        out_specs=[
            pl.BlockSpec((BLK_ROWS, COLS), lambda i: (i, 0)),
            pl.BlockSpec((1, 1, 128), lambda i: (i, 0, 0)),
        ],
        out_shape=[
            jax.ShapeDtypeStruct((ROWS, COLS), jnp.float32),
            jax.ShapeDtypeStruct((NBLK, 1, 128), jnp.float32),
        ],
    )(a, b, v, xy, xy)
    return s0, stats


def _lperm(x, idx):
    dn = lax.GatherDimensionNumbers(
        offset_dims=(), collapsed_slice_dims=(0,), start_index_map=(0,))
    return lax.gather(x, idx.reshape(16, 1), dn, slice_sizes=(1,),
                      mode=lax.GatherScatterMode.PROMISE_IN_BOUNDS)


def _allmax(x, lane):
    for s in (1, 2, 4, 8):
        x = jnp.maximum(x, _lperm(x, lane ^ s))
    return x


def _allmin(x, lane):
    for s in (1, 2, 4, 8):
        x = jnp.minimum(x, _lperm(x, lane ^ s))
    return x


def _sc_topk_body(s0_hbm, vals_hbm, idx_hbm, data_v, vrow_v, irow_v):
    wid = lax.axis_index("s") * 2 + lax.axis_index("c")
    pltpu.sync_copy(s0_hbm.at[pl.ds(wid * SHARD, SHARD)], data_v)

    lane = lax.iota(jnp.int32, 16)
    fneg = jnp.full((16,), -3.0e38, jnp.float32)

    def group_scan(g_base):
        def body(i, carry):
            v1, v2, i1 = carry
            base = g_base + i * 16
            x = data_v[pl.ds(base, 16)]
            idxv = base + lane
            upd = x > v1
            v2 = jnp.where(upd, v1, jnp.where(x > v2, x, v2))
            i1 = jnp.where(upd, idxv, i1)
            v1 = jnp.where(upd, x, v1)
            return v1, v2, i1

        return lax.fori_loop(
            0, GVEC, body, (fneg, fneg, jnp.full((16,), BIG, jnp.int32)))

    sg = fneg
    for g in range(NGRP):
        v1, _, _ = group_scan(jnp.int32(g * GRP))
        sg = jnp.where(lane == g, _allmax(v1, lane), sg)

    vrow = fneg
    irow = jnp.zeros((16,), jnp.int32)
    for t in range(STEPS):
        mtv = _allmax(sg, lane)
        gsv = _allmin(jnp.where(sg == mtv, lane, BIG), lane)
        v1, v2, i1 = group_scan(gsv[0] * GRP)
        mgv = _allmax(v1, lane)
        posv = _allmin(jnp.where(v1 == mgv, i1, BIG), lane)
        pos = posv[0]
        wl = pos & 15
        wbase = pos - wl
        w = data_v[pl.ds(wbase, 16)]
        data_v[pl.ds(wbase, 16)] = jnp.where(lane == wl, NEG, w)
        sg_new = _allmax(jnp.where(lane == wl, v2, v1), lane)
        sg = jnp.where(lane == gsv, sg_new, sg)
        vrow = jnp.where(lane == t, mgv, vrow)
        irow = jnp.where(lane == t, posv + wid * SHARD, irow)

    vrow_v[...] = vrow
    irow_v[...] = irow
    pltpu.sync_copy(vrow_v, vals_hbm.at[wid])
    pltpu.sync_copy(irow_v, idx_hbm.at[wid])


def _sc_topk(s0_flat):
    mesh = plsc.VectorSubcoreMesh(core_axis_name="c", subcore_axis_name="s")
    kfn = functools.partial(
        pl.kernel,
        mesh=mesh,
        out_type=[
            jax.ShapeDtypeStruct((NWORK, 16), jnp.float32),
            jax.ShapeDtypeStruct((NWORK, 16), jnp.int32),
        ],
        scratch_types=[
            pltpu.VMEM((SHARD,), jnp.float32),
            pltpu.VMEM((16,), jnp.float32),
            pltpu.VMEM((16,), jnp.int32),
        ],
    )(_sc_topk_body)
    return kfn(s0_flat)


def _merge_body(v_ref, i_ref, st_ref, ti_ref, lp_ref):
    V = v_ref[...]
    I = i_ref[...]
    st = st_ref[...]
    i2 = jax.lax.broadcasted_iota(jnp.int32, (NBLK, 1, 128), 2)
    mcol = jnp.max(jnp.where(i2 == 0, st, NEG), axis=2, keepdims=True)
    scol = jnp.sum(jnp.where(i2 == 1, st, 0.0), axis=2, keepdims=True)
    M = jnp.max(mcol)
    S = jnp.sum(jnp.exp(mcol - M) * scol)
    lane16 = jax.lax.broadcasted_iota(jnp.int32, (1, 16), 1)
    ti = jnp.zeros((1, 16), jnp.int32)
    lp = jnp.zeros((1, 16), jnp.float32)
    for t in range(STEPS):
        cur = jnp.max(V)
        pick = jnp.min(jnp.where(V == cur, I, BIG))
        V = jnp.where((V == cur) & (I == pick), NEG, V)
        lp = jnp.where(lane16 == t, cur - M - jnp.log(S), lp)
        ti = jnp.where(lane16 == t, pick, ti)
        S = S - jnp.exp(cur - M)
    ti_ref[...] = ti
    lp_ref[...] = lp


def _merge(cand_v, cand_i, stats):
    return pl.pallas_call(
        _merge_body,
        out_shape=[
            jax.ShapeDtypeStruct((1, STEPS), jnp.int32),
            jax.ShapeDtypeStruct((1, STEPS), jnp.float32),
        ],
    )(cand_v, cand_i, stats)


def kernel(static, dynamic, station_num_lim, W, v):
    s0, stats = _scores(static, W, v)
    cand_v, cand_i = _sc_topk(s0.reshape(-1))
    tour_idx, tour_logp = _merge(cand_v, cand_i, stats)
    return tour_idx, tour_logp


# speedup vs baseline: 26.8403x; 1.0846x over previous
import functools

import jax
import jax.numpy as jnp
from jax import lax
from jax.experimental import pallas as pl
from jax.experimental.pallas import tpu as pltpu
from jax.experimental.pallas import tpu_sc as plsc

N = 1_000_000
NP = 1 << 20
ROWS, COLS = 1024, 1024
BLK_ROWS = 128
NBLK = ROWS // BLK_ROWS
H = 16
NEG = -1e30
STEPS = 16

NWORK = 32
SHARD = NP // NWORK
NGRP = 16
GRP = SHARD // NGRP
GVEC = GRP // 16
UNROLL = 4
GPB = BLK_ROWS * COLS // GRP
BIG = 0x7FFFFFFF


def _bf(x):
    return x.astype(jnp.bfloat16).astype(jnp.float32)


def _score_body(a_ref, b_ref, v_ref, x_ref, y_ref, s_ref, stat_ref, gmax_ref):
    bi = pl.program_id(0)
    x = _bf(x_ref[0])
    y = _bf(y_ref[0])
    s = jnp.zeros_like(x)
    for j in range(H):
        z = a_ref[j] * x + b_ref[j] * y
        s = s + v_ref[j] * _bf(jnp.tanh(z))
    r = jax.lax.broadcasted_iota(jnp.int32, (BLK_ROWS, COLS), 0)
    c = jax.lax.broadcasted_iota(jnp.int32, (BLK_ROWS, COLS), 1)
    gidx = (bi * BLK_ROWS + r) * COLS + c
    s = jnp.where(gidx < N, s, NEG)
    s_ref[...] = s
    gmax_ref[...] = jnp.max(s.reshape(GPB, 2, COLS), axis=(1, 2)).reshape(1, 1, GPB)
    m = jnp.max(s)
    se = jnp.sum(jnp.exp(s - m))
    lane = jax.lax.broadcasted_iota(jnp.int32, (1, 1, 128), 2)
    stat_ref[...] = jnp.where(lane == 0, m, jnp.where(lane == 1, se, 0.0))


def _scores(static, W, v):
    xy = jnp.concatenate(
        [static.reshape(2, N), jnp.zeros((2, NP - N), jnp.float32)], axis=1
    ).reshape(2, ROWS, COLS)
    a = jax.lax.reduce_precision(W[:, 0], exponent_bits=8, mantissa_bits=7)
    b = jax.lax.reduce_precision(W[:, 1], exponent_bits=8, mantissa_bits=7)
    v = jax.lax.reduce_precision(v, exponent_bits=8, mantissa_bits=7)
    smem_spec = pl.BlockSpec(memory_space=pltpu.SMEM)
    s0, stats, gmax = pl.pallas_call(
        _score_body,
        grid=(NBLK,),
        in_specs=[
            smem_spec,
            smem_spec,
            smem_spec,
            pl.BlockSpec((1, BLK_ROWS, COLS), lambda i: (0, i, 0)),
            pl.BlockSpec((1, BLK_ROWS, COLS), lambda i: (1, i, 0)),
        ],
        out_specs=[
            pl.BlockSpec((BLK_ROWS, COLS), lambda i: (i, 0)),
            pl.BlockSpec((1, 1, 128), lambda i: (i, 0, 0)),
            pl.BlockSpec((1, 1, GPB), lambda i: (i, 0, 0)),
        ],
        out_shape=[
            jax.ShapeDtypeStruct((ROWS, COLS), jnp.float32),
            jax.ShapeDtypeStruct((NBLK, 1, 128), jnp.float32),
            jax.ShapeDtypeStruct((NBLK, 1, GPB), jnp.float32),
        ],
    )(a, b, v, xy, xy)
    return s0, stats, gmax


def _lperm(x, idx):
    dn = lax.GatherDimensionNumbers(
        offset_dims=(), collapsed_slice_dims=(0,), start_index_map=(0,))
    return lax.gather(x, idx.reshape(16, 1), dn, slice_sizes=(1,),
                      mode=lax.GatherScatterMode.PROMISE_IN_BOUNDS)


def _allmax(x, lane):
    for s in (1, 2, 4, 8):
        x = jnp.maximum(x, _lperm(x, lane ^ s))
    return x


def _allmin(x, lane):
    for s in (1, 2, 4, 8):
        x = jnp.minimum(x, _lperm(x, lane ^ s))
    return x


def _sc_topk_body(s0_hbm, gmax_hbm, vals_hbm, idx_hbm, data_v, sg_v, vrow_v,
                  irow_v):
    wid = lax.axis_index("s") * 2 + lax.axis_index("c")
    pltpu.sync_copy(s0_hbm.at[pl.ds(wid * SHARD, SHARD)], data_v)
    pltpu.sync_copy(gmax_hbm.at[pl.ds(wid * NGRP, NGRP)], sg_v)

    lane = lax.iota(jnp.int32, 16)
    fneg = jnp.full((16,), -3.0e38, jnp.float32)
    sg = sg_v[...]

    def group_scan(g_base):
        def body(i, carry):
            v1, v2, i1 = carry
            for u in range(UNROLL):
                base = g_base + (i * UNROLL + u) * 16
                x = data_v[pl.ds(base, 16)]
                idxv = base + lane
                upd = x > v1
                v2 = jnp.where(upd, v1, jnp.where(x > v2, x, v2))
                i1 = jnp.where(upd, idxv, i1)
                v1 = jnp.where(upd, x, v1)
            return v1, v2, i1

        return lax.fori_loop(
            0, GVEC // UNROLL, body,
            (fneg, fneg, jnp.full((16,), BIG, jnp.int32)))

    vrow = fneg
    irow = jnp.zeros((16,), jnp.int32)
    for t in range(STEPS):
        mtv = _allmax(sg, lane)
        gsv = _allmin(jnp.where(sg == mtv, lane, BIG), lane)
        v1, v2, i1 = group_scan(gsv[0] * GRP)
        mgv = _allmax(v1, lane)
        posv = _allmin(jnp.where(v1 == mgv, i1, BIG), lane)
        pos = posv[0]
        wl = pos & 15
        wbase = pos - wl
        w = data_v[pl.ds(wbase, 16)]
        data_v[pl.ds(wbase, 16)] = jnp.where(lane == wl, NEG, w)
        sg_new = _allmax(jnp.where(lane == wl, v2, v1), lane)
        sg = jnp.where(lane == gsv, sg_new, sg)
        vrow = jnp.where(lane == t, mgv, vrow)
        irow = jnp.where(lane == t, posv + wid * SHARD, irow)

    vrow_v[...] = vrow
    irow_v[...] = irow
    pltpu.sync_copy(vrow_v, vals_hbm.at[wid])
    pltpu.sync_copy(irow_v, idx_hbm.at[wid])


def _sc_topk(s0_flat, gmax_flat):
    mesh = plsc.VectorSubcoreMesh(core_axis_name="c", subcore_axis_name="s")
    kfn = functools.partial(
        pl.kernel,
        mesh=mesh,
        out_type=[
            jax.ShapeDtypeStruct((NWORK, 16), jnp.float32),
            jax.ShapeDtypeStruct((NWORK, 16), jnp.int32),
        ],
        scratch_types=[
            pltpu.VMEM((SHARD,), jnp.float32),
            pltpu.VMEM((16,), jnp.float32),
            pltpu.VMEM((16,), jnp.float32),
            pltpu.VMEM((16,), jnp.int32),
        ],
    )(_sc_topk_body)
    return kfn(s0_flat, gmax_flat)


def _merge_body(v_ref, i_ref, st_ref, ti_ref, lp_ref):
    V = v_ref[...]
    I = i_ref[...]
    st = st_ref[...]
    i2 = jax.lax.broadcasted_iota(jnp.int32, (NBLK, 1, 128), 2)
    mcol = jnp.max(jnp.where(i2 == 0, st, NEG), axis=2, keepdims=True)
    scol = jnp.sum(jnp.where(i2 == 1, st, 0.0), axis=2, keepdims=True)
    M = jnp.max(mcol)
    S = jnp.sum(jnp.exp(mcol - M) * scol)
    lane16 = jax.lax.broadcasted_iota(jnp.int32, (1, 16), 1)
    ti = jnp.zeros((1, 16), jnp.int32)
    lp = jnp.zeros((1, 16), jnp.float32)
    for t in range(STEPS):
        cur = jnp.max(V)
        pick = jnp.min(jnp.where(V == cur, I, BIG))
        V = jnp.where((V == cur) & (I == pick), NEG, V)
        lp = jnp.where(lane16 == t, cur - M - jnp.log(S), lp)
        ti = jnp.where(lane16 == t, pick, ti)
        S = S - jnp.exp(cur - M)
    ti_ref[...] = ti
    lp_ref[...] = lp


def _merge(cand_v, cand_i, stats):
    return pl.pallas_call(
        _merge_body,
        out_shape=[
            jax.ShapeDtypeStruct((1, STEPS), jnp.int32),
            jax.ShapeDtypeStruct((1, STEPS), jnp.float32),
        ],
    )(cand_v, cand_i, stats)


def kernel(static, dynamic, station_num_lim, W, v):
    s0, stats, gmax = _scores(static, W, v)
    cand_v, cand_i = _sc_topk(s0.reshape(-1), gmax.reshape(-1))
    tour_idx, tour_logp = _merge(cand_v, cand_i, stats)
    return tour_idx, tour_logp
